# Initial kernel scaffold; baseline (speedup 1.0000x reference)
import jax
import jax.numpy as jnp
from jax.experimental import pallas as pl


def kernel(x, edge_index, weight_edge, weight_self, bias_edge, bias_self):
    raise NotImplementedError("write your pallas kernel here")


# speedup vs baseline: 1.1030x; 1.1030x over previous
import functools

import jax
import jax.numpy as jnp
from jax import lax
from jax.experimental import pallas as pl
from jax.experimental.pallas import tpu as pltpu
from jax.experimental.pallas import tpu_sc as plsc


def _tc_edge_t(w2, xs_t, bias_t, n_edges, d, block):
    grid = n_edges // block
    d2 = d * d

    def body(w_ref, xs_ref, b_ref, o_ref):
        w3 = w_ref[...].reshape(d, d, block)
        xs = xs_ref[...]
        o_ref[...] = (w3 * xs[None, :, :]).sum(axis=1) + b_ref[...]

    return pl.pallas_call(
        body,
        grid=(grid,),
        in_specs=[
            pl.BlockSpec((d2, block), lambda i: (0, i)),
            pl.BlockSpec((d, block), lambda i: (0, i)),
            pl.BlockSpec((d, block), lambda i: (0, i)),
        ],
        out_specs=pl.BlockSpec((d, block), lambda i: (0, i)),
        out_shape=jax.ShapeDtypeStruct((d, n_edges), jnp.float32),
    )(w2, xs_t, bias_t)


def _tc_self_combine_t(w2, x_t, bias_t, agg_t, n_nodes, d):
    d2 = d * d

    def body(w_ref, x_ref, b_ref, a_ref, o_ref):
        w3 = w_ref[...].reshape(d, d, n_nodes)
        x = x_ref[...]
        own = (w3 * x[None, :, :]).sum(axis=1) + b_ref[...]
        o_ref[...] = jnp.maximum(own + a_ref[...], 0.0)

    return pl.pallas_call(
        body,
        grid=(1,),
        in_specs=[
            pl.BlockSpec((d2, n_nodes), lambda i: (0, 0)),
            pl.BlockSpec((d, n_nodes), lambda i: (0, 0)),
            pl.BlockSpec((d, n_nodes), lambda i: (0, 0)),
            pl.BlockSpec((d, n_nodes), lambda i: (0, 0)),
        ],
        out_specs=pl.BlockSpec((d, n_nodes), lambda i: (0, 0)),
        out_shape=jax.ShapeDtypeStruct((d, n_nodes), jnp.float32),
    )(w2, x_t, bias_t, agg_t)


def kernel(x, edge_index, weight_edge, weight_self, bias_edge, bias_self):
    n_nodes, d = x.shape
    n_edges = edge_index.shape[1]

    w2e = weight_edge.transpose(1, 2, 0).reshape(d * d, n_edges)
    w2s = weight_self.transpose(1, 2, 0).reshape(d * d, n_nodes)
    x_t = x.T
    be_t = bias_edge.T
    bs_t = bias_self.T

    src = edge_index[0]
    dst = edge_index[1]
    xs_t = jnp.take(x, src, axis=0).T
    contrib_t = _tc_edge_t(w2e, xs_t, be_t, n_edges, d, block=3200)
    agg = jnp.zeros((n_nodes, d), jnp.float32).at[dst].add(contrib_t.T)
    out_t = _tc_self_combine_t(w2s, x_t, bs_t, agg.T, n_nodes, d)
    return out_t.T


# speedup vs baseline: 1.5885x; 1.4402x over previous
import functools

import jax
import jax.numpy as jnp
from jax import lax
from jax.experimental import pallas as pl
from jax.experimental.pallas import tpu as pltpu
from jax.experimental.pallas import tpu_sc as plsc

NC = 2
NS = 16
CH = 128


def _tc_edge_t(w2, xs_t, bias_t, n_edges, d, block):
    grid = n_edges // block
    d2 = d * d

    def body(w_ref, xs_ref, b_ref, o_ref):
        w3 = w_ref[...].reshape(d, d, block)
        xs = xs_ref[...]
        o_ref[...] = (w3 * xs[None, :, :]).sum(axis=1) + b_ref[...]

    return pl.pallas_call(
        body,
        grid=(grid,),
        in_specs=[
            pl.BlockSpec((d2, block), lambda i: (0, i)),
            pl.BlockSpec((d, block), lambda i: (0, i)),
            pl.BlockSpec((d, block), lambda i: (0, i)),
        ],
        out_specs=pl.BlockSpec((d, block), lambda i: (0, i)),
        out_shape=jax.ShapeDtypeStruct((d, n_edges), jnp.float32),
    )(w2, xs_t, bias_t)


def _tc_self_t(w2, x_t, bias_t, n_nodes, d):
    d2 = d * d

    def body(w_ref, x_ref, b_ref, o_ref):
        w3 = w_ref[...].reshape(d, d, n_nodes)
        x = x_ref[...]
        o_ref[...] = (w3 * x[None, :, :]).sum(axis=1) + b_ref[...]

    return pl.pallas_call(
        body,
        grid=(1,),
        in_specs=[
            pl.BlockSpec((d2, n_nodes), lambda i: (0, 0)),
            pl.BlockSpec((d, n_nodes), lambda i: (0, 0)),
            pl.BlockSpec((d, n_nodes), lambda i: (0, 0)),
        ],
        out_specs=pl.BlockSpec((d, n_nodes), lambda i: (0, 0)),
        out_shape=jax.ShapeDtypeStruct((d, n_nodes), jnp.float32),
    )(w2, x_t, bias_t)


def _sc_gather(x_flat, src_idx, n_nodes, n_edges, d):
    nchunk = n_edges // CH
    nw = NC * NS
    jmax = -(-nchunk // nw)
    mesh = plsc.VectorSubcoreMesh(
        core_axis_name="c", subcore_axis_name="s",
        num_cores=NC, num_subcores=NS)

    @functools.partial(
        pl.kernel,
        mesh=mesh,
        out_type=jax.ShapeDtypeStruct((d * n_edges,), jnp.float32),
        scratch_types=(
            [pltpu.VMEM((CH,), jnp.int32)]
            + [pltpu.VMEM((CH,), jnp.int32) for _ in range(d)]
            + [pltpu.VMEM((CH,), jnp.float32) for _ in range(d)]
            + [pltpu.SemaphoreType.DMA, pltpu.SemaphoreType.DMA]
        ),
    )
    def k(x_hbm, src_hbm, out_hbm, *sc):
        sidx = sc[0]
        fidx = sc[1:1 + d]
        rows = sc[1 + d:1 + 2 * d]
        sem_g, sem_o = sc[1 + 2 * d], sc[2 + 2 * d]
        wid = lax.axis_index("s") * NC + lax.axis_index("c")

        def body(j, carry):
            cid = j * nw + wid

            @pl.when(cid < nchunk)
            def _():
                base = cid * CH
                pltpu.sync_copy(src_hbm.at[pl.ds(base, CH)], sidx)
                for k8 in range(CH // 16):
                    sv = sidx[pl.ds(k8 * 16, 16)]
                    for i in range(d):
                        fidx[i][pl.ds(k8 * 16, 16)] = sv + (i * n_nodes)
                gs = [
                    pltpu.async_copy(x_hbm.at[fidx[i]], rows[i], sem_g)
                    for i in range(d)
                ]
                for g in gs:
                    g.wait()
                os_ = [
                    pltpu.async_copy(
                        rows[i],
                        out_hbm.at[pl.ds(i * n_edges + base, CH)],
                        sem_o,
                    )
                    for i in range(d)
                ]
                for o in os_:
                    o.wait()

            return carry

        lax.fori_loop(0, jmax, body, 0)

    return k(x_flat, src_idx)


def _sc_scatter_combine(contrib_flat, fdst_flat, self_flat,
                        n_nodes, n_edges_pad, d):
    nchunk = n_edges_pad // CH
    jmax = nchunk // NS
    words_per_tile = (n_nodes * d) // NS
    mesh = plsc.VectorSubcoreMesh(
        core_axis_name="c", subcore_axis_name="s",
        num_cores=1, num_subcores=NS)

    @functools.partial(
        pl.kernel,
        mesh=mesh,
        out_type=jax.ShapeDtypeStruct((d * n_nodes,), jnp.float32),
        scratch_types=(
            [pltpu.VMEM((CH,), jnp.int32) for _ in range(d)]
            + [pltpu.VMEM((CH,), jnp.float32) for _ in range(d)]
            + [pltpu.VMEM((words_per_tile,), jnp.float32),
               pltpu.VMEM((words_per_tile,), jnp.float32),
               pltpu.VMEM_SHARED((d * n_nodes,), jnp.float32),
               pltpu.SemaphoreType.DMA,
               pltpu.SemaphoreType.DMA]
        ),
    )
    def k(co_hbm, fdst_hbm, self_hbm, out_hbm, *sc):
        fidx = sc[0:d]
        slab = sc[d:2 * d]
        accv, selfv, acc = sc[2 * d], sc[2 * d + 1], sc[2 * d + 2]
        sem_r, sem_f = sc[2 * d + 3], sc[2 * d + 4]
        t = lax.axis_index("s")

        def zero_row(i, carry):
            accv[pl.ds(i * 16, 16)] = jnp.zeros((16,), jnp.float32)
            return carry

        lax.fori_loop(0, words_per_tile // 16, zero_row, 0)
        pltpu.sync_copy(accv, acc.at[pl.ds(t * words_per_tile,
                                           words_per_tile)])
        plsc.subcore_barrier()

        def body(j, carry):
            cid = j * NS + t
            base = cid * CH
            rs = [
                pltpu.async_copy(
                    fdst_hbm.at[pl.ds(i * n_edges_pad + base, CH)],
                    fidx[i], sem_r)
                for i in range(d)
            ] + [
                pltpu.async_copy(
                    co_hbm.at[pl.ds(i * n_edges_pad + base, CH)],
                    slab[i], sem_r)
                for i in range(d)
            ]
            for r in rs:
                r.wait()
            for i in range(d):
                pltpu.sync_copy(slab[i], acc.at[fidx[i]], add=True)
            return carry

        lax.fori_loop(0, jmax, body, 0)
        plsc.subcore_barrier()

        w0 = t * words_per_tile
        pltpu.sync_copy(acc.at[pl.ds(w0, words_per_tile)], accv)
        pltpu.sync_copy(self_hbm.at[pl.ds(w0, words_per_tile)], selfv)

        def cg(g, carry):
            v = accv[pl.ds(g * 16, 16)] + selfv[pl.ds(g * 16, 16)]
            accv[pl.ds(g * 16, 16)] = jnp.maximum(v, 0.0)
            return carry

        lax.fori_loop(0, words_per_tile // 16, cg, 0)
        pltpu.sync_copy(accv, out_hbm.at[pl.ds(w0, words_per_tile)])

    return k(contrib_flat, fdst_flat, self_flat)


def kernel(x, edge_index, weight_edge, weight_self, bias_edge, bias_self):
    n_nodes, d = x.shape
    n_edges = edge_index.shape[1]

    w2e = weight_edge.transpose(1, 2, 0).reshape(d * d, n_edges)
    w2s = weight_self.transpose(1, 2, 0).reshape(d * d, n_nodes)
    x_t = x.T
    be_t = bias_edge.T
    bs_t = bias_self.T

    src = edge_index[0]
    dst = edge_index[1]

    self_t = _tc_self_t(w2s, x_t, bs_t, n_nodes, d)
    self_flat = self_t.reshape(d * n_nodes)

    xs_t = jnp.take(x, src, axis=0).T
    contrib_t = _tc_edge_t(w2e, xs_t, be_t, n_edges, d, block=3200)

    e_pad = -(-n_edges // (CH * NS)) * (CH * NS)
    pad = e_pad - n_edges
    contrib_pad = jnp.concatenate(
        [contrib_t, jnp.zeros((d, pad), jnp.float32)], axis=1)
    dst_pad = jnp.concatenate([dst, jnp.zeros((pad,), jnp.int32)])
    offs = (jnp.arange(d, dtype=jnp.int32) * n_nodes)[:, None]
    fdst_flat = (dst_pad[None, :] + offs).reshape(d * e_pad)

    out_flat = _sc_scatter_combine(contrib_pad.reshape(d * e_pad),
                                   fdst_flat, self_flat,
                                   n_nodes, e_pad, d)
    return out_flat.reshape(d, n_nodes).T


# speedup vs baseline: 1.7768x; 1.1185x over previous
import functools

import jax
import jax.numpy as jnp
from jax import lax
from jax.experimental import pallas as pl
from jax.experimental.pallas import tpu as pltpu
from jax.experimental.pallas import tpu_sc as plsc

NC = 2
NS = 16
CH = 128


def _tc_edge_t(w2, xs_t, bias_t, n_edges, d, block):
    grid = n_edges // block
    d2 = d * d

    def body(w_ref, xs_ref, b_ref, o_ref):
        w3 = w_ref[...].reshape(d, d, block)
        xs = xs_ref[...]
        o_ref[...] = (w3 * xs[None, :, :]).sum(axis=1) + b_ref[...]

    return pl.pallas_call(
        body,
        grid=(grid,),
        in_specs=[
            pl.BlockSpec((d2, block), lambda i: (0, i)),
            pl.BlockSpec((d, block), lambda i: (0, i)),
            pl.BlockSpec((d, block), lambda i: (0, i)),
        ],
        out_specs=pl.BlockSpec((d, block), lambda i: (0, i)),
        out_shape=jax.ShapeDtypeStruct((d, n_edges), jnp.float32),
    )(w2, xs_t, bias_t)


def _tc_self_t(w2, x_t, bias_t, n_nodes, d):
    d2 = d * d

    def body(w_ref, x_ref, b_ref, o_ref):
        w3 = w_ref[...].reshape(d, d, n_nodes)
        x = x_ref[...]
        o_ref[...] = (w3 * x[None, :, :]).sum(axis=1) + b_ref[...]

    return pl.pallas_call(
        body,
        grid=(1,),
        in_specs=[
            pl.BlockSpec((d2, n_nodes), lambda i: (0, 0)),
            pl.BlockSpec((d, n_nodes), lambda i: (0, 0)),
            pl.BlockSpec((d, n_nodes), lambda i: (0, 0)),
        ],
        out_specs=pl.BlockSpec((d, n_nodes), lambda i: (0, 0)),
        out_shape=jax.ShapeDtypeStruct((d, n_nodes), jnp.float32),
    )(w2, x_t, bias_t)


def _sc_gather(x_flat, src_idx, n_nodes, n_edges, d):
    nchunk = n_edges // CH
    nw = NC * NS
    jmax = -(-nchunk // nw)
    mesh = plsc.VectorSubcoreMesh(
        core_axis_name="c", subcore_axis_name="s",
        num_cores=NC, num_subcores=NS)

    @functools.partial(
        pl.kernel,
        mesh=mesh,
        out_type=jax.ShapeDtypeStruct((d * n_edges,), jnp.float32),
        scratch_types=(
            [pltpu.VMEM((CH,), jnp.int32)]
            + [pltpu.VMEM((CH,), jnp.int32) for _ in range(d)]
            + [pltpu.VMEM((CH,), jnp.float32) for _ in range(d)]
            + [pltpu.SemaphoreType.DMA, pltpu.SemaphoreType.DMA]
        ),
    )
    def k(x_hbm, src_hbm, out_hbm, *sc):
        sidx = sc[0]
        fidx = sc[1:1 + d]
        rows = sc[1 + d:1 + 2 * d]
        sem_g, sem_o = sc[1 + 2 * d], sc[2 + 2 * d]
        wid = lax.axis_index("s") * NC + lax.axis_index("c")

        def body(j, carry):
            cid = j * nw + wid

            @pl.when(cid < nchunk)
            def _():
                base = cid * CH
                pltpu.sync_copy(src_hbm.at[pl.ds(base, CH)], sidx)
                for k8 in range(CH // 16):
                    sv = sidx[pl.ds(k8 * 16, 16)]
                    for i in range(d):
                        fidx[i][pl.ds(k8 * 16, 16)] = sv + (i * n_nodes)
                gs = [
                    pltpu.async_copy(x_hbm.at[fidx[i]], rows[i], sem_g)
                    for i in range(d)
                ]
                for g in gs:
                    g.wait()
                os_ = [
                    pltpu.async_copy(
                        rows[i],
                        out_hbm.at[pl.ds(i * n_edges + base, CH)],
                        sem_o,
                    )
                    for i in range(d)
                ]
                for o in os_:
                    o.wait()

            return carry

        lax.fori_loop(0, jmax, body, 0)

    return k(x_flat, src_idx)


def _sc_scatter_combine(contrib_flat, fdst_flat, self_flat,
                        n_nodes, n_edges_pad, d):
    nchunk = n_edges_pad // CH
    jmax = nchunk // NS
    words_per_tile = (n_nodes * d) // NS
    mesh = plsc.VectorSubcoreMesh(
        core_axis_name="c", subcore_axis_name="s",
        num_cores=1, num_subcores=NS)

    @functools.partial(
        pl.kernel,
        mesh=mesh,
        out_type=jax.ShapeDtypeStruct((d * n_nodes,), jnp.float32),
        scratch_types=(
            [pltpu.VMEM((CH,), jnp.int32) for _ in range(d)]
            + [pltpu.VMEM((CH,), jnp.float32) for _ in range(d)]
            + [pltpu.VMEM((words_per_tile,), jnp.float32),
               pltpu.VMEM((words_per_tile,), jnp.float32),
               pltpu.VMEM_SHARED((d * n_nodes,), jnp.float32),
               pltpu.SemaphoreType.DMA,
               pltpu.SemaphoreType.DMA]
        ),
    )
    def k(co_hbm, fdst_hbm, self_hbm, out_hbm, *sc):
        fidx = sc[0:d]
        slab = sc[d:2 * d]
        accv, selfv, acc = sc[2 * d], sc[2 * d + 1], sc[2 * d + 2]
        sem_r, sem_f = sc[2 * d + 3], sc[2 * d + 4]
        t = lax.axis_index("s")

        def zero_row(i, carry):
            accv[pl.ds(i * 16, 16)] = jnp.zeros((16,), jnp.float32)
            return carry

        lax.fori_loop(0, words_per_tile // 16, zero_row, 0)
        pltpu.sync_copy(accv, acc.at[pl.ds(t * words_per_tile,
                                           words_per_tile)])
        plsc.subcore_barrier()

        def body(j, carry):
            cid = j * NS + t
            base = cid * CH
            rs = [
                pltpu.async_copy(
                    fdst_hbm.at[pl.ds(i * n_edges_pad + base, CH)],
                    fidx[i], sem_r)
                for i in range(d)
            ] + [
                pltpu.async_copy(
                    co_hbm.at[pl.ds(i * n_edges_pad + base, CH)],
                    slab[i], sem_r)
                for i in range(d)
            ]
            for r in rs:
                r.wait()
            ads = [
                pltpu.async_copy(slab[i], acc.at[fidx[i]], sem_f, add=True)
                for i in range(d)
            ]
            for a in ads:
                a.wait()
            return carry

        lax.fori_loop(0, jmax, body, 0)
        plsc.subcore_barrier()

        w0 = t * words_per_tile
        pltpu.sync_copy(acc.at[pl.ds(w0, words_per_tile)], accv)
        pltpu.sync_copy(self_hbm.at[pl.ds(w0, words_per_tile)], selfv)

        def cg(g, carry):
            v = accv[pl.ds(g * 16, 16)] + selfv[pl.ds(g * 16, 16)]
            accv[pl.ds(g * 16, 16)] = jnp.maximum(v, 0.0)
            return carry

        lax.fori_loop(0, words_per_tile // 16, cg, 0)
        pltpu.sync_copy(accv, out_hbm.at[pl.ds(w0, words_per_tile)])

    return k(contrib_flat, fdst_flat, self_flat)


def kernel(x, edge_index, weight_edge, weight_self, bias_edge, bias_self):
    n_nodes, d = x.shape
    n_edges = edge_index.shape[1]

    w2e = weight_edge.transpose(1, 2, 0).reshape(d * d, n_edges)
    w2s = weight_self.transpose(1, 2, 0).reshape(d * d, n_nodes)
    x_t = x.T
    be_t = bias_edge.T
    bs_t = bias_self.T

    src = edge_index[0]
    dst = edge_index[1]

    self_t = _tc_self_t(w2s, x_t, bs_t, n_nodes, d)
    self_flat = self_t.reshape(d * n_nodes)

    xs_t = jnp.take(x, src, axis=0).T
    contrib_t = _tc_edge_t(w2e, xs_t, be_t, n_edges, d, block=3200)

    e_pad = -(-n_edges // (CH * NS)) * (CH * NS)
    pad = e_pad - n_edges
    contrib_pad = jnp.concatenate(
        [contrib_t, jnp.zeros((d, pad), jnp.float32)], axis=1)
    dst_pad = jnp.concatenate([dst, jnp.zeros((pad,), jnp.int32)])
    offs = (jnp.arange(d, dtype=jnp.int32) * n_nodes)[:, None]
    fdst_flat = (dst_pad[None, :] + offs).reshape(d * e_pad)

    out_flat = _sc_scatter_combine(contrib_pad.reshape(d * e_pad),
                                   fdst_flat, self_flat,
                                   n_nodes, e_pad, d)
    return out_flat.reshape(d, n_nodes).T


# speedup vs baseline: 2.7205x; 1.5311x over previous
import functools

import jax
import jax.numpy as jnp
from jax import lax
from jax.experimental import pallas as pl
from jax.experimental.pallas import tpu as pltpu
from jax.experimental.pallas import tpu_sc as plsc

NC = 2
NS = 16
CH = 128


def _tc_edge_t(w2, xs_t, bias_t, n_edges, d, block):
    grid = n_edges // block
    d2 = d * d

    def body(w_ref, xs_ref, b_ref, o_ref):
        w3 = w_ref[...].reshape(d, d, block)
        xs = xs_ref[...]
        o_ref[...] = (w3 * xs[None, :, :]).sum(axis=1) + b_ref[...]

    return pl.pallas_call(
        body,
        grid=(grid,),
        in_specs=[
            pl.BlockSpec((d2, block), lambda i: (0, i)),
            pl.BlockSpec((d, block), lambda i: (0, i)),
            pl.BlockSpec((d, block), lambda i: (0, i)),
        ],
        out_specs=pl.BlockSpec((d, block), lambda i: (0, i)),
        out_shape=jax.ShapeDtypeStruct((d, n_edges), jnp.float32),
    )(w2, xs_t, bias_t)


def _tc_self_t(w2, x_t, bias_t, n_nodes, d):
    d2 = d * d

    def body(w_ref, x_ref, b_ref, o_ref):
        w3 = w_ref[...].reshape(d, d, n_nodes)
        x = x_ref[...]
        o_ref[...] = (w3 * x[None, :, :]).sum(axis=1) + b_ref[...]

    return pl.pallas_call(
        body,
        grid=(1,),
        in_specs=[
            pl.BlockSpec((d2, n_nodes), lambda i: (0, 0)),
            pl.BlockSpec((d, n_nodes), lambda i: (0, 0)),
            pl.BlockSpec((d, n_nodes), lambda i: (0, 0)),
        ],
        out_specs=pl.BlockSpec((d, n_nodes), lambda i: (0, 0)),
        out_shape=jax.ShapeDtypeStruct((d, n_nodes), jnp.float32),
    )(w2, x_t, bias_t)


def _sc_gather(x_flat, src_idx, n_nodes, n_edges, d):
    nchunk = n_edges // CH
    nw = NC * NS
    jmax = -(-nchunk // nw)
    mesh = plsc.VectorSubcoreMesh(
        core_axis_name="c", subcore_axis_name="s",
        num_cores=NC, num_subcores=NS)

    @functools.partial(
        pl.kernel,
        mesh=mesh,
        out_type=jax.ShapeDtypeStruct((d * n_edges,), jnp.float32),
        scratch_types=(
            [pltpu.VMEM((CH,), jnp.int32)]
            + [pltpu.VMEM((CH,), jnp.int32) for _ in range(d)]
            + [pltpu.VMEM((CH,), jnp.float32) for _ in range(d)]
            + [pltpu.SemaphoreType.DMA, pltpu.SemaphoreType.DMA]
        ),
    )
    def k(x_hbm, src_hbm, out_hbm, *sc):
        sidx = sc[0]
        fidx = sc[1:1 + d]
        rows = sc[1 + d:1 + 2 * d]
        sem_g, sem_o = sc[1 + 2 * d], sc[2 + 2 * d]
        wid = lax.axis_index("s") * NC + lax.axis_index("c")

        def body(j, carry):
            cid = j * nw + wid

            @pl.when(cid < nchunk)
            def _():
                base = cid * CH
                pltpu.sync_copy(src_hbm.at[pl.ds(base, CH)], sidx)
                for k8 in range(CH // 16):
                    sv = sidx[pl.ds(k8 * 16, 16)]
                    for i in range(d):
                        fidx[i][pl.ds(k8 * 16, 16)] = sv + (i * n_nodes)
                gs = [
                    pltpu.async_copy(x_hbm.at[fidx[i]], rows[i], sem_g)
                    for i in range(d)
                ]
                for g in gs:
                    g.wait()
                os_ = [
                    pltpu.async_copy(
                        rows[i],
                        out_hbm.at[pl.ds(i * n_edges + base, CH)],
                        sem_o,
                    )
                    for i in range(d)
                ]
                for o in os_:
                    o.wait()

            return carry

        lax.fori_loop(0, jmax, body, 0)

    return k(x_flat, src_idx)


def _sc_scatter_combine(contrib_flat, dst_pad, self_flat,
                        n_nodes, n_edges_pad, d):
    nchunk = n_edges_pad // CH
    jmax = nchunk // NS
    words_per_tile = (n_nodes * d) // NS
    mesh = plsc.VectorSubcoreMesh(
        core_axis_name="c", subcore_axis_name="s",
        num_cores=1, num_subcores=NS)

    @functools.partial(
        pl.kernel,
        mesh=mesh,
        out_type=jax.ShapeDtypeStruct((d * n_nodes,), jnp.float32),
        scratch_types=(
            [pltpu.VMEM((CH,), jnp.int32)]
            + [pltpu.VMEM((CH,), jnp.int32) for _ in range(d)]
            + [pltpu.VMEM((CH,), jnp.float32) for _ in range(d)]
            + [pltpu.VMEM((words_per_tile,), jnp.float32),
               pltpu.VMEM((words_per_tile,), jnp.float32),
               pltpu.VMEM_SHARED((d * n_nodes,), jnp.float32),
               pltpu.SemaphoreType.DMA,
               pltpu.SemaphoreType.DMA]
        ),
    )
    def k(co_hbm, dst_hbm, self_hbm, out_hbm, *sc):
        sidx = sc[0]
        fidx = sc[1:1 + d]
        slab = sc[1 + d:1 + 2 * d]
        accv, selfv, acc = sc[1 + 2 * d], sc[2 + 2 * d], sc[3 + 2 * d]
        sem_r, sem_f = sc[4 + 2 * d], sc[5 + 2 * d]
        t = lax.axis_index("s")

        def zero_row(i, carry):
            accv[pl.ds(i * 16, 16)] = jnp.zeros((16,), jnp.float32)
            return carry

        lax.fori_loop(0, words_per_tile // 16, zero_row, 0)
        pltpu.sync_copy(accv, acc.at[pl.ds(t * words_per_tile,
                                           words_per_tile)])
        plsc.subcore_barrier()

        def body(j, carry):
            cid = j * NS + t
            base = cid * CH
            rs = [
                pltpu.async_copy(
                    co_hbm.at[pl.ds(i * n_edges_pad + base, CH)],
                    slab[i], sem_r)
                for i in range(d)
            ]
            pltpu.sync_copy(dst_hbm.at[pl.ds(base, CH)], sidx)
            for k8 in range(CH // 16):
                sv = sidx[pl.ds(k8 * 16, 16)]
                for i in range(d):
                    fidx[i][pl.ds(k8 * 16, 16)] = sv + (i * n_nodes)
            for r in rs:
                r.wait()
            ads = [
                pltpu.async_copy(slab[i], acc.at[fidx[i]], sem_f, add=True)
                for i in range(d)
            ]
            for a in ads:
                a.wait()
            return carry

        lax.fori_loop(0, jmax, body, 0)
        plsc.subcore_barrier()

        w0 = t * words_per_tile
        pltpu.sync_copy(acc.at[pl.ds(w0, words_per_tile)], accv)
        pltpu.sync_copy(self_hbm.at[pl.ds(w0, words_per_tile)], selfv)

        def cg(g, carry):
            v = accv[pl.ds(g * 16, 16)] + selfv[pl.ds(g * 16, 16)]
            accv[pl.ds(g * 16, 16)] = jnp.maximum(v, 0.0)
            return carry

        lax.fori_loop(0, words_per_tile // 16, cg, 0)
        pltpu.sync_copy(accv, out_hbm.at[pl.ds(w0, words_per_tile)])

    return k(contrib_flat, dst_pad, self_flat)


def kernel(x, edge_index, weight_edge, weight_self, bias_edge, bias_self):
    n_nodes, d = x.shape
    n_edges = edge_index.shape[1]

    w2e = weight_edge.transpose(1, 2, 0).reshape(d * d, n_edges)
    w2s = weight_self.transpose(1, 2, 0).reshape(d * d, n_nodes)
    x_t = x.T
    be_t = bias_edge.T
    bs_t = bias_self.T

    src = edge_index[0]
    dst = edge_index[1]

    x_flat = x_t.reshape(d * n_nodes)
    self_t = _tc_self_t(w2s, x_t, bs_t, n_nodes, d)
    self_flat = self_t.reshape(d * n_nodes)

    xsrc_flat = _sc_gather(x_flat, src, n_nodes, n_edges, d)
    xs_t = xsrc_flat.reshape(d, n_edges)
    contrib_t = _tc_edge_t(w2e, xs_t, be_t, n_edges, d, block=3200)

    e_pad = -(-n_edges // (CH * NS)) * (CH * NS)
    pad = e_pad - n_edges
    contrib_pad = jnp.concatenate(
        [contrib_t, jnp.zeros((d, pad), jnp.float32)], axis=1)
    dst_pad = jnp.concatenate([dst, jnp.zeros((pad,), jnp.int32)])

    out_flat = _sc_scatter_combine(contrib_pad.reshape(d * e_pad),
                                   dst_pad, self_flat,
                                   n_nodes, e_pad, d)
    return out_flat.reshape(d, n_nodes).T


# speedup vs baseline: 2.8756x; 1.0570x over previous
import functools

import jax
import jax.numpy as jnp
from jax import lax
from jax.experimental import pallas as pl
from jax.experimental.pallas import tpu as pltpu
from jax.experimental.pallas import tpu_sc as plsc

NC = 2
NS = 16
CH = 128


def _tc_edge_t(w2, xs_t, bias_t, n_edges, d, block):
    grid = n_edges // block
    d2 = d * d

    def body(w_ref, xs_ref, b_ref, o_ref):
        w3 = w_ref[...].reshape(d, d, block)
        xs = xs_ref[...]
        o_ref[...] = (w3 * xs[None, :, :]).sum(axis=1) + b_ref[...]

    return pl.pallas_call(
        body,
        grid=(grid,),
        in_specs=[
            pl.BlockSpec((d2, block), lambda i: (0, i)),
            pl.BlockSpec((d, block), lambda i: (0, i)),
            pl.BlockSpec((d, block), lambda i: (0, i)),
        ],
        out_specs=pl.BlockSpec((d, block), lambda i: (0, i)),
        out_shape=jax.ShapeDtypeStruct((d, n_edges), jnp.float32),
    )(w2, xs_t, bias_t)


def _tc_self_t(w2, x_t, bias_t, n_nodes, d):
    d2 = d * d

    def body(w_ref, x_ref, b_ref, o_ref):
        w3 = w_ref[...].reshape(d, d, n_nodes)
        x = x_ref[...]
        o_ref[...] = (w3 * x[None, :, :]).sum(axis=1) + b_ref[...]

    return pl.pallas_call(
        body,
        grid=(1,),
        in_specs=[
            pl.BlockSpec((d2, n_nodes), lambda i: (0, 0)),
            pl.BlockSpec((d, n_nodes), lambda i: (0, 0)),
            pl.BlockSpec((d, n_nodes), lambda i: (0, 0)),
        ],
        out_specs=pl.BlockSpec((d, n_nodes), lambda i: (0, 0)),
        out_shape=jax.ShapeDtypeStruct((d, n_nodes), jnp.float32),
    )(w2, x_t, bias_t)


def _sc_gather(x_flat, src_idx, n_nodes, n_edges, d):
    nchunk = n_edges // CH
    nw = NC * NS
    jmax = -(-nchunk // nw)
    mesh = plsc.VectorSubcoreMesh(
        core_axis_name="c", subcore_axis_name="s",
        num_cores=NC, num_subcores=NS)

    @functools.partial(
        pl.kernel,
        mesh=mesh,
        out_type=jax.ShapeDtypeStruct((d * n_edges,), jnp.float32),
        scratch_types=(
            [pltpu.VMEM((CH,), jnp.int32)]
            + [pltpu.VMEM((CH,), jnp.int32) for _ in range(d)]
            + [pltpu.VMEM((CH,), jnp.float32) for _ in range(d)]
            + [pltpu.SemaphoreType.DMA, pltpu.SemaphoreType.DMA]
        ),
    )
    def k(x_hbm, src_hbm, out_hbm, *sc):
        sidx = sc[0]
        fidx = sc[1:1 + d]
        rows = sc[1 + d:1 + 2 * d]
        sem_g, sem_o = sc[1 + 2 * d], sc[2 + 2 * d]
        wid = lax.axis_index("s") * NC + lax.axis_index("c")

        def body(j, carry):
            cid = j * nw + wid

            @pl.when(cid < nchunk)
            def _():
                base = cid * CH
                pltpu.sync_copy(src_hbm.at[pl.ds(base, CH)], sidx)
                for k8 in range(CH // 16):
                    sv = sidx[pl.ds(k8 * 16, 16)]
                    for i in range(d):
                        fidx[i][pl.ds(k8 * 16, 16)] = sv + (i * n_nodes)
                gs = [
                    pltpu.async_copy(x_hbm.at[fidx[i]], rows[i], sem_g)
                    for i in range(d)
                ]
                for g in gs:
                    g.wait()
                os_ = [
                    pltpu.async_copy(
                        rows[i],
                        out_hbm.at[pl.ds(i * n_edges + base, CH)],
                        sem_o,
                    )
                    for i in range(d)
                ]
                for o in os_:
                    o.wait()

            return carry

        lax.fori_loop(0, jmax, body, 0)

    return k(x_flat, src_idx)


def _sc_scatter_combine(contrib_flat, dst_pad, self_flat,
                        n_nodes, n_edges_pad, d):
    nchunk = n_edges_pad // CH
    jmax = -(-nchunk // NS)
    words_per_tile = (n_nodes * d) // NS
    mesh = plsc.VectorSubcoreMesh(
        core_axis_name="c", subcore_axis_name="s",
        num_cores=1, num_subcores=NS)

    @functools.partial(
        pl.kernel,
        mesh=mesh,
        out_type=jax.ShapeDtypeStruct((d * n_nodes,), jnp.float32),
        scratch_types=(
            [pltpu.VMEM((CH,), jnp.int32)]
            + [pltpu.VMEM((CH,), jnp.int32) for _ in range(d)]
            + [pltpu.VMEM((CH,), jnp.float32) for _ in range(d)]
            + [pltpu.VMEM((words_per_tile,), jnp.float32),
               pltpu.VMEM((words_per_tile,), jnp.float32),
               pltpu.VMEM_SHARED((d * n_nodes,), jnp.float32),
               pltpu.SemaphoreType.DMA,
               pltpu.SemaphoreType.DMA]
        ),
    )
    def k(co_hbm, dst_hbm, self_hbm, out_hbm, *sc):
        sidx = sc[0]
        fidx = sc[1:1 + d]
        slab = sc[1 + d:1 + 2 * d]
        accv, selfv, acc = sc[1 + 2 * d], sc[2 + 2 * d], sc[3 + 2 * d]
        sem_r, sem_f = sc[4 + 2 * d], sc[5 + 2 * d]
        t = lax.axis_index("s")

        def zero_row(i, carry):
            accv[pl.ds(i * 16, 16)] = jnp.zeros((16,), jnp.float32)
            return carry

        lax.fori_loop(0, words_per_tile // 16, zero_row, 0)
        pltpu.sync_copy(accv, acc.at[pl.ds(t * words_per_tile,
                                           words_per_tile)])
        plsc.subcore_barrier()

        def body(j, carry):
            cid = j * NS + t

            @pl.when(cid < nchunk)
            def _():
                base = cid * CH
                rs = [
                    pltpu.async_copy(
                        co_hbm.at[pl.ds(i * n_edges_pad + base, CH)],
                        slab[i], sem_r)
                    for i in range(d)
                ]
                pltpu.sync_copy(dst_hbm.at[pl.ds(base, CH)], sidx)
                for k8 in range(CH // 16):
                    sv = sidx[pl.ds(k8 * 16, 16)]
                    for i in range(d):
                        fidx[i][pl.ds(k8 * 16, 16)] = sv + (i * n_nodes)
                for r in rs:
                    r.wait()
                ads = [
                    pltpu.async_copy(slab[i], acc.at[fidx[i]], sem_f,
                                     add=True)
                    for i in range(d)
                ]
                for a in ads:
                    a.wait()

            return carry

        lax.fori_loop(0, jmax, body, 0)
        plsc.subcore_barrier()

        w0 = t * words_per_tile
        pltpu.sync_copy(acc.at[pl.ds(w0, words_per_tile)], accv)
        pltpu.sync_copy(self_hbm.at[pl.ds(w0, words_per_tile)], selfv)

        def cg(g, carry):
            v = accv[pl.ds(g * 16, 16)] + selfv[pl.ds(g * 16, 16)]
            accv[pl.ds(g * 16, 16)] = jnp.maximum(v, 0.0)
            return carry

        lax.fori_loop(0, words_per_tile // 16, cg, 0)
        pltpu.sync_copy(accv, out_hbm.at[pl.ds(w0, words_per_tile)])

    return k(contrib_flat, dst_pad, self_flat)


def kernel(x, edge_index, weight_edge, weight_self, bias_edge, bias_self):
    n_nodes, d = x.shape
    n_edges = edge_index.shape[1]

    w2e = weight_edge.transpose(1, 2, 0).reshape(d * d, n_edges)
    w2s = weight_self.transpose(1, 2, 0).reshape(d * d, n_nodes)
    x_t = x.T
    be_t = bias_edge.T
    bs_t = bias_self.T

    src = edge_index[0]
    dst = edge_index[1]

    x_flat = x_t.reshape(d * n_nodes)
    self_t = _tc_self_t(w2s, x_t, bs_t, n_nodes, d)
    self_flat = self_t.reshape(d * n_nodes)

    xsrc_flat = _sc_gather(x_flat, src, n_nodes, n_edges, d)
    xs_t = xsrc_flat.reshape(d, n_edges)
    contrib_t = _tc_edge_t(w2e, xs_t, be_t, n_edges, d, block=6400)

    out_flat = _sc_scatter_combine(contrib_t.reshape(d * n_edges),
                                   dst, self_flat,
                                   n_nodes, n_edges, d)
    return out_flat.reshape(d, n_nodes).T
